# baseline (speedup 1.0000x reference)
import jax
import jax.numpy as jnp
from jax import lax
from jax.experimental import pallas as pl
from jax.experimental.pallas import tpu as pltpu
from jax.experimental.pallas import tpu_sc as plsc

T_OUT = 8192
T_IN = 4096
L = 16
NS = 16
BR = 256
PT = T_OUT // NS
SLICE = T_IN // NS


def _tc_argmax_body(x_ref, o_ref):
    o_ref[...] = jnp.argmax(x_ref[...], axis=-1).astype(jnp.int32)


def _sc_bincount_body(idx_hbm, out_hbm, idxbuf, hist, tmp, acc, shared, csem):
    sid = lax.axis_index("s")
    lane = lax.iota(jnp.int32, L)
    zeros_i = jnp.zeros((L,), jnp.int32)
    ones_i = jnp.ones((L,), jnp.int32)

    cp = pltpu.make_async_copy(idx_hbm.at[pl.ds(sid * PT, PT)], idxbuf, csem)
    cp.start()

    def _zero(i, _):
        hist[pl.ds(i * L, L)] = zeros_i
        return 0
    lax.fori_loop(0, T_IN // L, _zero, 0)
    cp.wait()

    def _bump(v, _):
        iv = idxbuf[pl.ds(v * L, L)]
        for k in range(L):
            plsc.addupdate_scatter(hist, [iv], ones_i, mask=lane == k)
        return 0
    lax.fori_loop(0, PT // L, _bump, 0)

    pltpu.sync_copy(hist, shared.at[sid])
    plsc.subcore_barrier()

    colbase = sid * SLICE
    descs = [
        pltpu.make_async_copy(
            shared.at[t, pl.ds(colbase, SLICE)], tmp.at[t], csem
        )
        for t in range(NS)
    ]
    for d in descs:
        d.start()
    for d in descs:
        d.wait()

    for i in range(SLICE // L):
        s = pl.ds(i * L, L)
        v = tmp[0, s]
        for t in range(1, NS):
            v = v + tmp[t, s]
        acc[s] = v

    pltpu.sync_copy(acc, out_hbm.at[pl.ds(colbase, SLICE)])


@jax.jit
def kernel(att_ws):
    idx = pl.pallas_call(
        _tc_argmax_body,
        grid=(T_OUT // BR,),
        in_specs=[pl.BlockSpec((BR, T_IN), lambda i: (i, 0))],
        out_specs=pl.BlockSpec((BR,), lambda i: (i,)),
        out_shape=jax.ShapeDtypeStruct((T_OUT,), jnp.int32),
    )(att_ws)
    mesh = plsc.VectorSubcoreMesh(
        core_axis_name="c", subcore_axis_name="s", num_cores=1, num_subcores=NS
    )
    out = pl.kernel(
        _sc_bincount_body,
        out_type=jax.ShapeDtypeStruct((T_IN,), jnp.int32),
        mesh=mesh,
        compiler_params=pltpu.CompilerParams(needs_layout_passes=False),
        scratch_types=[
            pltpu.VMEM((PT,), jnp.int32),
            pltpu.VMEM((T_IN,), jnp.int32),
            pltpu.VMEM((NS, SLICE), jnp.int32),
            pltpu.VMEM((SLICE,), jnp.int32),
            pltpu.VMEM_SHARED((NS, T_IN), jnp.int32),
            pltpu.SemaphoreType.DMA,
        ],
    )(idx)
    return out
